# baseline (speedup 1.0000x reference)
import functools
import math

import jax
import jax.numpy as jnp
from jax import lax
from jax.experimental import pallas as pl
from jax.experimental.pallas import tpu as pltpu

D_MODEL = 768
NUM_HEADS = 12
HEAD_DIM = 64
D_FF = 3072
S_LEN = 4096
NUM_LOCAL = 614
L_PAD = 640
LBLK = 128
N_LBLK = 5


def _ln(xb, g, b, eps=1e-5):
    mu = jnp.mean(xb, axis=-1, keepdims=True)
    var = jnp.mean((xb - mu) ** 2, axis=-1, keepdims=True)
    return (xb - mu) * jax.lax.rsqrt(var + eps) * g + b


def _kv_body(x_ref, g_ref, be_ref, wk_ref, bk_ref, wv_ref, bv_ref, k_ref, v_ref):
    xn = _ln(x_ref[...], g_ref[...], be_ref[...]).astype(jnp.bfloat16)
    k_ref[...] = (jnp.dot(xn, wk_ref[...], preferred_element_type=jnp.float32)
                  + bk_ref[...]).astype(jnp.bfloat16)
    v_ref[...] = (jnp.dot(xn, wv_ref[...], preferred_element_type=jnp.float32)
                  + bv_ref[...]).astype(jnp.bfloat16)


def _kv_proj(x, g1, be1, Wk, bk, Wv, bv):
    blk = 512
    grid = S_LEN // blk
    return pl.pallas_call(
        _kv_body,
        grid=(grid,),
        in_specs=[
            pl.BlockSpec((blk, D_MODEL), lambda i: (i, 0)),
            pl.BlockSpec((1, D_MODEL), lambda i: (0, 0)),
            pl.BlockSpec((1, D_MODEL), lambda i: (0, 0)),
            pl.BlockSpec((D_MODEL, D_MODEL), lambda i: (0, 0)),
            pl.BlockSpec((1, D_MODEL), lambda i: (0, 0)),
            pl.BlockSpec((D_MODEL, D_MODEL), lambda i: (0, 0)),
            pl.BlockSpec((1, D_MODEL), lambda i: (0, 0)),
        ],
        out_specs=[
            pl.BlockSpec((blk, D_MODEL), lambda i: (i, 0)),
            pl.BlockSpec((blk, D_MODEL), lambda i: (i, 0)),
        ],
        out_shape=[
            jax.ShapeDtypeStruct((S_LEN, D_MODEL), jnp.bfloat16),
            jax.ShapeDtypeStruct((S_LEN, D_MODEL), jnp.bfloat16),
        ],
    )(x, g1, be1, Wk, bk, Wv, bv)


def _topk_body(vcol_ref, vrow_ref, idx_ref):
    ii = lax.broadcasted_iota(jnp.int32, (S_LEN, 1), 0)
    vcol = jnp.where(ii == 0, -1.0, vcol_ref[...])

    cblk = 512
    nchunk = S_LEN // cblk

    def rank_step(c, cnt):
        vj = vrow_ref[:, pl.ds(c * cblk, cblk)]
        jj = c * cblk + lax.broadcasted_iota(jnp.int32, (1, cblk), 1)
        vj = jnp.where(jj == 0, -1.0, vj)
        beats = (vj > vcol) | ((vj == vcol) & (jj < ii))
        return cnt + jnp.sum(beats.astype(jnp.float32), axis=1, keepdims=True)

    rank = lax.fori_loop(0, nchunk, rank_step, jnp.zeros((S_LEN, 1), jnp.float32))

    ii_f = ii.astype(jnp.float32)
    for rc in range(L_PAD // 128):
        rr = (rc * 128 + lax.broadcasted_iota(jnp.int32, (1, 128), 1)).astype(jnp.float32)
        eq = (rank == rr).astype(jnp.float32)
        idx = jnp.sum(eq * ii_f, axis=0, keepdims=True)
        idx_ref[:, pl.ds(rc * 128, 128)] = idx.astype(jnp.int32)


def _topk(cls_col, cls_row):
    return pl.pallas_call(
        _topk_body,
        in_specs=[
            pl.BlockSpec(memory_space=pltpu.VMEM),
            pl.BlockSpec(memory_space=pltpu.VMEM),
        ],
        out_specs=pl.BlockSpec(memory_space=pltpu.VMEM),
        out_shape=jax.ShapeDtypeStruct((1, L_PAD), jnp.int32),
    )(cls_col, cls_row)


def _q_body(idx_ref, x_ref, g_ref, be_ref, wq_ref, bq_ref, q_ref, gat_ref):
    lb = pl.program_id(0)

    def gather_step(i, _):
        row = x_ref[pl.ds(idx_ref[0, lb * LBLK + i], 1), :]
        gat_ref[pl.ds(i, 1), :] = row
        return 0

    lax.fori_loop(0, LBLK, gather_step, 0)
    xn = _ln(gat_ref[...], g_ref[...], be_ref[...]).astype(jnp.bfloat16)
    scale = 1.0 / math.sqrt(HEAD_DIM)
    q_ref[...] = ((jnp.dot(xn, wq_ref[...], preferred_element_type=jnp.float32)
                   + bq_ref[...]) * scale).astype(jnp.bfloat16)


def _q_proj(top_idx, x, g1, be1, Wq, bq):
    return pl.pallas_call(
        _q_body,
        grid=(N_LBLK,),
        in_specs=[
            pl.BlockSpec(memory_space=pltpu.SMEM),
            pl.BlockSpec((S_LEN, D_MODEL), lambda i: (0, 0)),
            pl.BlockSpec((1, D_MODEL), lambda i: (0, 0)),
            pl.BlockSpec((1, D_MODEL), lambda i: (0, 0)),
            pl.BlockSpec((D_MODEL, D_MODEL), lambda i: (0, 0)),
            pl.BlockSpec((1, D_MODEL), lambda i: (0, 0)),
        ],
        out_specs=pl.BlockSpec((LBLK, D_MODEL), lambda i: (i, 0)),
        out_shape=jax.ShapeDtypeStruct((L_PAD, D_MODEL), jnp.bfloat16),
        scratch_shapes=[pltpu.VMEM((LBLK, D_MODEL), jnp.float32)],
    )(top_idx, x, g1, be1, Wq, bq)


ABLK = 64


def _attn_body(q_ref, k_ref, v_ref, probs_ref, ao_ref):
    for h in range(NUM_HEADS):
        qh = q_ref[:, h * HEAD_DIM:(h + 1) * HEAD_DIM]
        kh = k_ref[:, h * HEAD_DIM:(h + 1) * HEAD_DIM]
        vh = v_ref[:, h * HEAD_DIM:(h + 1) * HEAD_DIM]
        s = lax.dot_general(qh, kh, (((1,), (1,)), ((), ())),
                            preferred_element_type=jnp.float32)
        m = jnp.max(s, axis=1, keepdims=True)
        p = jnp.exp(s - m)
        denom = jnp.sum(p, axis=1, keepdims=True)
        probs = p / denom
        probs_ref[0, h] = probs
        ao_ref[:, h * HEAD_DIM:(h + 1) * HEAD_DIM] = jnp.dot(
            probs.astype(jnp.bfloat16), vh, preferred_element_type=jnp.float32)


def _attention(q, k, v):
    return pl.pallas_call(
        _attn_body,
        grid=(L_PAD // ABLK,),
        in_specs=[
            pl.BlockSpec((ABLK, D_MODEL), lambda l: (l, 0)),
            pl.BlockSpec((S_LEN, D_MODEL), lambda l: (0, 0)),
            pl.BlockSpec((S_LEN, D_MODEL), lambda l: (0, 0)),
        ],
        out_specs=[
            pl.BlockSpec((1, NUM_HEADS, ABLK, S_LEN), lambda l: (0, 0, l, 0)),
            pl.BlockSpec((ABLK, D_MODEL), lambda l: (l, 0)),
        ],
        out_shape=[
            jax.ShapeDtypeStruct((1, NUM_HEADS, NUM_LOCAL, S_LEN), jnp.float32),
            jax.ShapeDtypeStruct((L_PAD, D_MODEL), jnp.float32),
        ],
    )(q, k, v)


def _oproj_body(a_ref, wo_ref, bo_ref, o_ref):
    o_ref[...] = jnp.dot(a_ref[...].astype(jnp.bfloat16), wo_ref[...],
                         preferred_element_type=jnp.float32) + bo_ref[...]


def _out_proj(attn_out, Wo, bo):
    return pl.pallas_call(
        _oproj_body,
        grid=(N_LBLK,),
        in_specs=[
            pl.BlockSpec((LBLK, D_MODEL), lambda i: (i, 0)),
            pl.BlockSpec((D_MODEL, D_MODEL), lambda i: (0, 0)),
            pl.BlockSpec((1, D_MODEL), lambda i: (0, 0)),
        ],
        out_specs=pl.BlockSpec((LBLK, D_MODEL), lambda i: (i, 0)),
        out_shape=jax.ShapeDtypeStruct((L_PAD, D_MODEL), jnp.float32),
    )(attn_out, Wo, bo)


def _scatter_body(idx_ref, alpha_ref, x_ref, lo_ref, out_ref):
    out_ref[...] = x_ref[...]
    alpha = alpha_ref[0]

    def step(i, _):
        idx = idx_ref[0, i]
        out_ref[pl.ds(idx, 1), :] = (out_ref[pl.ds(idx, 1), :]
                                     + alpha * lo_ref[pl.ds(i, 1), :])
        return 0

    lax.fori_loop(0, NUM_LOCAL, step, 0)


def _scatter(top_idx, alpha, x, local_out):
    return pl.pallas_call(
        _scatter_body,
        in_specs=[
            pl.BlockSpec(memory_space=pltpu.SMEM),
            pl.BlockSpec(memory_space=pltpu.SMEM),
            pl.BlockSpec(memory_space=pltpu.VMEM),
            pl.BlockSpec(memory_space=pltpu.VMEM),
        ],
        out_specs=pl.BlockSpec(memory_space=pltpu.VMEM),
        out_shape=jax.ShapeDtypeStruct((S_LEN, D_MODEL), jnp.float32),
    )(top_idx, alpha, x, local_out)


def _ffn_body(x_ref, g_ref, be_ref, w1_ref, b1_ref, w2_ref, b2_ref, o_ref):
    xb = x_ref[...]
    xn = _ln(xb, g_ref[...], be_ref[...]).astype(jnp.bfloat16)
    h = jnp.dot(xn, w1_ref[...], preferred_element_type=jnp.float32) + b1_ref[...]
    h = 0.5 * h * (1.0 + lax.erf(h * (1.0 / math.sqrt(2.0))))
    ff = jnp.dot(h.astype(jnp.bfloat16), w2_ref[...],
                 preferred_element_type=jnp.float32) + b2_ref[...]
    o_ref[...] = xb + ff


def _ffn(x_local, g2, be2, W1, b1, W2, b2):
    blk = 256
    grid = S_LEN // blk
    return pl.pallas_call(
        _ffn_body,
        grid=(grid,),
        in_specs=[
            pl.BlockSpec((blk, D_MODEL), lambda i: (i, 0)),
            pl.BlockSpec((1, D_MODEL), lambda i: (0, 0)),
            pl.BlockSpec((1, D_MODEL), lambda i: (0, 0)),
            pl.BlockSpec((D_MODEL, D_FF), lambda i: (0, 0)),
            pl.BlockSpec((1, D_FF), lambda i: (0, 0)),
            pl.BlockSpec((D_FF, D_MODEL), lambda i: (0, 0)),
            pl.BlockSpec((1, D_MODEL), lambda i: (0, 0)),
        ],
        out_specs=pl.BlockSpec((blk, D_MODEL), lambda i: (i, 0)),
        out_shape=jax.ShapeDtypeStruct((S_LEN, D_MODEL), jnp.float32),
    )(x_local, g2, be2, W1, b1, W2, b2)


def kernel(x, attention_rollout, Wq, bq, Wk, bk, Wv, bv, Wo, bo, W1, b1, W2, b2, g1, be1, g2, be2, alpha):
    B = x.shape[0]
    x2 = x.reshape(S_LEN, D_MODEL)
    cls = attention_rollout[0, 0, :]
    cls_col = cls.reshape(S_LEN, 1)
    cls_row = cls.reshape(1, S_LEN)
    r = lambda a: a.reshape(1, -1)

    bf = lambda a: a.astype(jnp.bfloat16)
    k, v = _kv_proj(x2, r(g1), r(be1), bf(Wk), r(bk), bf(Wv), r(bv))
    top_idx = _topk(cls_col, cls_row)
    q = _q_proj(top_idx, x2, r(g1), r(be1), bf(Wq), r(bq))
    probs, attn_out = _attention(q, k, v)
    local_out = _out_proj(attn_out, bf(Wo), r(bo))
    x_local = _scatter(top_idx, alpha, x2, local_out)
    x_final = _ffn(x_local, r(g2), r(be2), bf(W1), r(b1), bf(W2), r(b2))

    return x_final.reshape(B, S_LEN, D_MODEL), probs


# speedup vs baseline: 1.1835x; 1.1835x over previous
import functools
import math

import jax
import jax.numpy as jnp
from jax import lax
from jax.experimental import pallas as pl
from jax.experimental.pallas import tpu as pltpu

D_MODEL = 768
NUM_HEADS = 12
HEAD_DIM = 64
D_FF = 3072
S_LEN = 4096
NUM_LOCAL = 614
L_PAD = 640
LBLK = 128
N_LBLK = 5


def _ln(xb, g, b, eps=1e-5):
    mu = jnp.mean(xb, axis=-1, keepdims=True)
    var = jnp.mean((xb - mu) ** 2, axis=-1, keepdims=True)
    return (xb - mu) * jax.lax.rsqrt(var + eps) * g + b


def _kv_body(x_ref, g_ref, be_ref, wk_ref, bk_ref, wv_ref, bv_ref, k_ref, v_ref):
    xn = _ln(x_ref[...], g_ref[...], be_ref[...]).astype(jnp.bfloat16)
    k_ref[...] = (jnp.dot(xn, wk_ref[...], preferred_element_type=jnp.float32)
                  + bk_ref[...]).astype(jnp.bfloat16)
    v_ref[...] = (jnp.dot(xn, wv_ref[...], preferred_element_type=jnp.float32)
                  + bv_ref[...]).astype(jnp.bfloat16)


def _kv_proj(x, g1, be1, Wk, bk, Wv, bv):
    blk = 512
    grid = S_LEN // blk
    return pl.pallas_call(
        _kv_body,
        grid=(grid,),
        in_specs=[
            pl.BlockSpec((blk, D_MODEL), lambda i: (i, 0)),
            pl.BlockSpec((1, D_MODEL), lambda i: (0, 0)),
            pl.BlockSpec((1, D_MODEL), lambda i: (0, 0)),
            pl.BlockSpec((D_MODEL, D_MODEL), lambda i: (0, 0)),
            pl.BlockSpec((1, D_MODEL), lambda i: (0, 0)),
            pl.BlockSpec((D_MODEL, D_MODEL), lambda i: (0, 0)),
            pl.BlockSpec((1, D_MODEL), lambda i: (0, 0)),
        ],
        out_specs=[
            pl.BlockSpec((blk, D_MODEL), lambda i: (i, 0)),
            pl.BlockSpec((blk, D_MODEL), lambda i: (i, 0)),
        ],
        out_shape=[
            jax.ShapeDtypeStruct((S_LEN, D_MODEL), jnp.bfloat16),
            jax.ShapeDtypeStruct((S_LEN, D_MODEL), jnp.bfloat16),
        ],
    )(x, g1, be1, Wk, bk, Wv, bv)


def _topk_body(vcol_ref, vrow_ref, idx_ref):
    ii = lax.broadcasted_iota(jnp.int32, (S_LEN, 1), 0)
    vcol = jnp.where(ii == 0, -1.0, vcol_ref[...])

    cblk = 512
    nchunk = S_LEN // cblk

    def rank_step(c, cnt):
        vj = vrow_ref[:, pl.ds(c * cblk, cblk)]
        jj = c * cblk + lax.broadcasted_iota(jnp.int32, (1, cblk), 1)
        vj = jnp.where(jj == 0, -1.0, vj)
        beats = (vj > vcol) | ((vj == vcol) & (jj < ii))
        return cnt + jnp.sum(beats.astype(jnp.float32), axis=1, keepdims=True)

    rank = lax.fori_loop(0, nchunk, rank_step, jnp.zeros((S_LEN, 1), jnp.float32))

    ii_f = ii.astype(jnp.float32)
    for rc in range(L_PAD // 128):
        rr = (rc * 128 + lax.broadcasted_iota(jnp.int32, (1, 128), 1)).astype(jnp.float32)
        eq = (rank == rr).astype(jnp.float32)
        idx = jnp.sum(eq * ii_f, axis=0, keepdims=True)
        idx_ref[:, pl.ds(rc * 128, 128)] = idx.astype(jnp.int32)


def _topk(cls_col, cls_row):
    return pl.pallas_call(
        _topk_body,
        in_specs=[
            pl.BlockSpec(memory_space=pltpu.VMEM),
            pl.BlockSpec(memory_space=pltpu.VMEM),
        ],
        out_specs=pl.BlockSpec(memory_space=pltpu.VMEM),
        out_shape=jax.ShapeDtypeStruct((1, L_PAD), jnp.int32),
    )(cls_col, cls_row)


def _q_body(idx_ref, x_ref, g_ref, be_ref, wq_ref, bq_ref, q_ref, gat_ref):
    lb = pl.program_id(0)

    def gather_step(i, _):
        row = x_ref[pl.ds(idx_ref[0, lb * LBLK + i], 1), :]
        gat_ref[pl.ds(i, 1), :] = row
        return 0

    lax.fori_loop(0, LBLK, gather_step, 0)
    xn = _ln(gat_ref[...], g_ref[...], be_ref[...]).astype(jnp.bfloat16)
    scale = 1.0 / math.sqrt(HEAD_DIM)
    q_ref[...] = ((jnp.dot(xn, wq_ref[...], preferred_element_type=jnp.float32)
                   + bq_ref[...]) * scale).astype(jnp.bfloat16)


def _q_proj(top_idx, x, g1, be1, Wq, bq):
    return pl.pallas_call(
        _q_body,
        grid=(N_LBLK,),
        in_specs=[
            pl.BlockSpec(memory_space=pltpu.SMEM),
            pl.BlockSpec((S_LEN, D_MODEL), lambda i: (0, 0)),
            pl.BlockSpec((1, D_MODEL), lambda i: (0, 0)),
            pl.BlockSpec((1, D_MODEL), lambda i: (0, 0)),
            pl.BlockSpec((D_MODEL, D_MODEL), lambda i: (0, 0)),
            pl.BlockSpec((1, D_MODEL), lambda i: (0, 0)),
        ],
        out_specs=pl.BlockSpec((LBLK, D_MODEL), lambda i: (i, 0)),
        out_shape=jax.ShapeDtypeStruct((L_PAD, D_MODEL), jnp.bfloat16),
        scratch_shapes=[pltpu.VMEM((LBLK, D_MODEL), jnp.float32)],
    )(top_idx, x, g1, be1, Wq, bq)


ABLK = 64


def _attn_body(q_ref, k_ref, v_ref, probs_ref, ao_ref):
    for h in range(NUM_HEADS):
        qh = q_ref[:, h * HEAD_DIM:(h + 1) * HEAD_DIM]
        kh = k_ref[:, h * HEAD_DIM:(h + 1) * HEAD_DIM]
        vh = v_ref[:, h * HEAD_DIM:(h + 1) * HEAD_DIM]
        s = lax.dot_general(qh, kh, (((1,), (1,)), ((), ())),
                            preferred_element_type=jnp.float32)
        m = jnp.max(s, axis=1, keepdims=True)
        p = jnp.exp(s - m)
        denom = jnp.sum(p, axis=1, keepdims=True)
        probs = p / denom
        probs_ref[h] = probs
        ao_ref[:, h * HEAD_DIM:(h + 1) * HEAD_DIM] = jnp.dot(
            probs.astype(jnp.bfloat16), vh, preferred_element_type=jnp.float32)


def _attention(q, k, v):
    return pl.pallas_call(
        _attn_body,
        grid=(L_PAD // ABLK,),
        in_specs=[
            pl.BlockSpec((ABLK, D_MODEL), lambda l: (l, 0)),
            pl.BlockSpec((S_LEN, D_MODEL), lambda l: (0, 0)),
            pl.BlockSpec((S_LEN, D_MODEL), lambda l: (0, 0)),
        ],
        out_specs=[
            pl.BlockSpec((NUM_HEADS, ABLK, S_LEN), lambda l: (0, l, 0)),
            pl.BlockSpec((ABLK, D_MODEL), lambda l: (l, 0)),
        ],
        out_shape=[
            jax.ShapeDtypeStruct((NUM_HEADS, NUM_LOCAL, S_LEN), jnp.float32),
            jax.ShapeDtypeStruct((L_PAD, D_MODEL), jnp.float32),
        ],
    )(q, k, v)


def _oproj_body(a_ref, wo_ref, bo_ref, o_ref):
    o_ref[...] = jnp.dot(a_ref[...].astype(jnp.bfloat16), wo_ref[...],
                         preferred_element_type=jnp.float32) + bo_ref[...]


def _out_proj(attn_out, Wo, bo):
    return pl.pallas_call(
        _oproj_body,
        grid=(N_LBLK,),
        in_specs=[
            pl.BlockSpec((LBLK, D_MODEL), lambda i: (i, 0)),
            pl.BlockSpec((D_MODEL, D_MODEL), lambda i: (0, 0)),
            pl.BlockSpec((1, D_MODEL), lambda i: (0, 0)),
        ],
        out_specs=pl.BlockSpec((LBLK, D_MODEL), lambda i: (i, 0)),
        out_shape=jax.ShapeDtypeStruct((L_PAD, D_MODEL), jnp.float32),
    )(attn_out, Wo, bo)


def _scatter_body(idx_ref, alpha_ref, x_ref, lo_ref, out_ref):
    out_ref[...] = x_ref[...]
    alpha = alpha_ref[0]

    def step(i, _):
        idx = idx_ref[0, i]
        out_ref[pl.ds(idx, 1), :] = (out_ref[pl.ds(idx, 1), :]
                                     + alpha * lo_ref[pl.ds(i, 1), :])
        return 0

    lax.fori_loop(0, NUM_LOCAL, step, 0)


def _scatter(top_idx, alpha, x, local_out):
    return pl.pallas_call(
        _scatter_body,
        in_specs=[
            pl.BlockSpec(memory_space=pltpu.SMEM),
            pl.BlockSpec(memory_space=pltpu.SMEM),
            pl.BlockSpec(memory_space=pltpu.VMEM),
            pl.BlockSpec(memory_space=pltpu.VMEM),
        ],
        out_specs=pl.BlockSpec(memory_space=pltpu.VMEM),
        out_shape=jax.ShapeDtypeStruct((S_LEN, D_MODEL), jnp.float32),
    )(top_idx, alpha, x, local_out)


def _ffn_body(x_ref, g_ref, be_ref, w1_ref, b1_ref, w2_ref, b2_ref, o_ref):
    xb = x_ref[...]
    xn = _ln(xb, g_ref[...], be_ref[...]).astype(jnp.bfloat16)
    h = jnp.dot(xn, w1_ref[...], preferred_element_type=jnp.float32) + b1_ref[...]
    h = 0.5 * h * (1.0 + lax.erf(h * (1.0 / math.sqrt(2.0))))
    ff = jnp.dot(h.astype(jnp.bfloat16), w2_ref[...],
                 preferred_element_type=jnp.float32) + b2_ref[...]
    o_ref[...] = xb + ff


def _ffn(x_local, g2, be2, W1, b1, W2, b2):
    blk = 256
    grid = S_LEN // blk
    return pl.pallas_call(
        _ffn_body,
        grid=(grid,),
        in_specs=[
            pl.BlockSpec((blk, D_MODEL), lambda i: (i, 0)),
            pl.BlockSpec((1, D_MODEL), lambda i: (0, 0)),
            pl.BlockSpec((1, D_MODEL), lambda i: (0, 0)),
            pl.BlockSpec((D_MODEL, D_FF), lambda i: (0, 0)),
            pl.BlockSpec((1, D_FF), lambda i: (0, 0)),
            pl.BlockSpec((D_FF, D_MODEL), lambda i: (0, 0)),
            pl.BlockSpec((1, D_MODEL), lambda i: (0, 0)),
        ],
        out_specs=pl.BlockSpec((blk, D_MODEL), lambda i: (i, 0)),
        out_shape=jax.ShapeDtypeStruct((S_LEN, D_MODEL), jnp.float32),
    )(x_local, g2, be2, W1, b1, W2, b2)


def kernel(x, attention_rollout, Wq, bq, Wk, bk, Wv, bv, Wo, bo, W1, b1, W2, b2, g1, be1, g2, be2, alpha):
    B = x.shape[0]
    x2 = x.reshape(S_LEN, D_MODEL)
    cls = attention_rollout[0, 0, :]
    cls_col = cls.reshape(S_LEN, 1)
    cls_row = cls.reshape(1, S_LEN)
    r = lambda a: a.reshape(1, -1)

    bf = lambda a: a.astype(jnp.bfloat16)
    k, v = _kv_proj(x2, r(g1), r(be1), bf(Wk), r(bk), bf(Wv), r(bv))
    top_idx = _topk(cls_col, cls_row)
    q = _q_proj(top_idx, x2, r(g1), r(be1), bf(Wq), r(bq))
    probs, attn_out = _attention(q, k, v)
    local_out = _out_proj(attn_out, bf(Wo), r(bo))
    x_local = _scatter(top_idx, alpha, x2, local_out)
    x_final = _ffn(x_local, r(g2), r(be2), bf(W1), r(b1), bf(W2), r(b2))

    return (x_final.reshape(B, S_LEN, D_MODEL),
            probs.reshape(B, NUM_HEADS, NUM_LOCAL, S_LEN))


# speedup vs baseline: 1.4330x; 1.2108x over previous
import functools
import math

import jax
import jax.numpy as jnp
from jax import lax
from jax.experimental import pallas as pl
from jax.experimental.pallas import tpu as pltpu

D_MODEL = 768
NUM_HEADS = 12
HEAD_DIM = 64
D_FF = 3072
S_LEN = 4096
NUM_LOCAL = 614
L_PAD = 640
LBLK = 128
N_LBLK = 5
BF = jnp.bfloat16


def _ln(xb, g, b, eps=1e-5):
    mu = jnp.mean(xb, axis=-1, keepdims=True)
    var = jnp.mean((xb - mu) ** 2, axis=-1, keepdims=True)
    return (xb - mu) * jax.lax.rsqrt(var + eps) * g + b


def _kv_body(x_ref, g_ref, be_ref, wk_ref, bk_ref, wv_ref, bv_ref,
             k_ref, v_ref, wkb, wvb):
    @pl.when(pl.program_id(0) == 0)
    def _():
        wkb[...] = wk_ref[...].astype(BF)
        wvb[...] = wv_ref[...].astype(BF)

    xn = _ln(x_ref[...], g_ref[...], be_ref[...]).astype(BF)
    k_ref[...] = (jnp.dot(xn, wkb[...], preferred_element_type=jnp.float32)
                  + bk_ref[...]).astype(BF)
    v_ref[...] = (jnp.dot(xn, wvb[...], preferred_element_type=jnp.float32)
                  + bv_ref[...]).astype(BF)


def _kv_proj(x, g1, be1, Wk, bk, Wv, bv):
    blk = 512
    return pl.pallas_call(
        _kv_body,
        grid=(S_LEN // blk,),
        in_specs=[
            pl.BlockSpec((blk, D_MODEL), lambda i: (i, 0)),
            pl.BlockSpec((D_MODEL,), lambda i: (0,)),
            pl.BlockSpec((D_MODEL,), lambda i: (0,)),
            pl.BlockSpec((D_MODEL, D_MODEL), lambda i: (0, 0)),
            pl.BlockSpec((D_MODEL,), lambda i: (0,)),
            pl.BlockSpec((D_MODEL, D_MODEL), lambda i: (0, 0)),
            pl.BlockSpec((D_MODEL,), lambda i: (0,)),
        ],
        out_specs=[
            pl.BlockSpec((blk, D_MODEL), lambda i: (i, 0)),
            pl.BlockSpec((blk, D_MODEL), lambda i: (i, 0)),
        ],
        out_shape=[
            jax.ShapeDtypeStruct((S_LEN, D_MODEL), BF),
            jax.ShapeDtypeStruct((S_LEN, D_MODEL), BF),
        ],
        scratch_shapes=[pltpu.VMEM((D_MODEL, D_MODEL), BF)] * 2,
    )(x, g1, be1, Wk, bk, Wv, bv)


def _topk_body(vcol_ref, vrow_ref, idx_ref):
    ii = lax.broadcasted_iota(jnp.int32, (S_LEN, 1), 0)
    vcol = jnp.where(ii == 0, -1.0, vcol_ref[...])

    cblk = 512
    nchunk = S_LEN // cblk

    def rank_step(c, cnt):
        vj = vrow_ref[:, pl.ds(c * cblk, cblk)]
        jj = c * cblk + lax.broadcasted_iota(jnp.int32, (1, cblk), 1)
        vj = jnp.where(jj == 0, -1.0, vj)
        beats = (vj > vcol) | ((vj == vcol) & (jj < ii))
        return cnt + jnp.sum(beats.astype(jnp.float32), axis=1, keepdims=True)

    rank = lax.fori_loop(0, nchunk, rank_step, jnp.zeros((S_LEN, 1), jnp.float32))

    ii_f = ii.astype(jnp.float32)
    for rc in range(L_PAD // 128):
        rr = (rc * 128 + lax.broadcasted_iota(jnp.int32, (1, 128), 1)).astype(jnp.float32)
        eq = (rank == rr).astype(jnp.float32)
        idx = jnp.sum(eq * ii_f, axis=0, keepdims=True)
        idx_ref[:, pl.ds(rc * 128, 128)] = idx.astype(jnp.int32)


def _topk(cls_col, cls_row):
    return pl.pallas_call(
        _topk_body,
        in_specs=[
            pl.BlockSpec(memory_space=pltpu.VMEM),
            pl.BlockSpec(memory_space=pltpu.VMEM),
        ],
        out_specs=pl.BlockSpec(memory_space=pltpu.VMEM),
        out_shape=jax.ShapeDtypeStruct((1, L_PAD), jnp.int32),
    )(cls_col, cls_row)


def _q_body(idx_ref, x_ref, g_ref, be_ref, wq_ref, bq_ref, q_ref, gat, wqb):
    lb = pl.program_id(0)

    @pl.when(lb == 0)
    def _():
        wqb[...] = wq_ref[...].astype(BF)

    def gather_step(i, _):
        gat[pl.ds(i, 1), :] = x_ref[pl.ds(idx_ref[0, lb * LBLK + i], 1), :]
        return 0

    lax.fori_loop(0, LBLK, gather_step, 0)
    xn = _ln(gat[...], g_ref[...], be_ref[...]).astype(BF)
    scale = 1.0 / math.sqrt(HEAD_DIM)
    q_ref[...] = ((jnp.dot(xn, wqb[...], preferred_element_type=jnp.float32)
                   + bq_ref[...]) * scale).astype(BF)


def _q_proj(top_idx, x, g1, be1, Wq, bq):
    return pl.pallas_call(
        _q_body,
        grid=(N_LBLK,),
        in_specs=[
            pl.BlockSpec(memory_space=pltpu.SMEM),
            pl.BlockSpec((S_LEN, D_MODEL), lambda i: (0, 0)),
            pl.BlockSpec((D_MODEL,), lambda i: (0,)),
            pl.BlockSpec((D_MODEL,), lambda i: (0,)),
            pl.BlockSpec((D_MODEL, D_MODEL), lambda i: (0, 0)),
            pl.BlockSpec((D_MODEL,), lambda i: (0,)),
        ],
        out_specs=pl.BlockSpec((LBLK, D_MODEL), lambda i: (i, 0)),
        out_shape=jax.ShapeDtypeStruct((L_PAD, D_MODEL), BF),
        scratch_shapes=[pltpu.VMEM((LBLK, D_MODEL), jnp.float32),
                        pltpu.VMEM((D_MODEL, D_MODEL), BF)],
    )(top_idx, x, g1, be1, Wq, bq)


ABLK = 80


def _attn_body(q_ref, k_ref, v_ref, wo_ref, bo_ref, probs_ref, lo_ref, wob):
    @pl.when(pl.program_id(0) == 0)
    def _():
        wob[...] = wo_ref[...].astype(BF)

    aos = []
    for h in range(NUM_HEADS):
        qh = q_ref[:, h * HEAD_DIM:(h + 1) * HEAD_DIM]
        kh = k_ref[:, h * HEAD_DIM:(h + 1) * HEAD_DIM]
        vh = v_ref[:, h * HEAD_DIM:(h + 1) * HEAD_DIM]
        s = lax.dot_general(qh, kh, (((1,), (1,)), ((), ())),
                            preferred_element_type=jnp.float32)
        m = jnp.max(s, axis=1, keepdims=True)
        p = jnp.exp(s - m)
        inv = 1.0 / jnp.sum(p, axis=1, keepdims=True)
        probs = p * inv
        probs_ref[h] = probs
        aos.append(jnp.dot(probs.astype(BF), vh,
                           preferred_element_type=jnp.float32))
    acc = jnp.concatenate(aos, axis=1)
    lo_ref[...] = jnp.dot(acc.astype(BF), wob[...],
                          preferred_element_type=jnp.float32) + bo_ref[...]


def _attention(q, k, v, Wo, bo):
    return pl.pallas_call(
        _attn_body,
        grid=(L_PAD // ABLK,),
        in_specs=[
            pl.BlockSpec((ABLK, D_MODEL), lambda l: (l, 0)),
            pl.BlockSpec((S_LEN, D_MODEL), lambda l: (0, 0)),
            pl.BlockSpec((S_LEN, D_MODEL), lambda l: (0, 0)),
            pl.BlockSpec((D_MODEL, D_MODEL), lambda l: (0, 0)),
            pl.BlockSpec((D_MODEL,), lambda l: (0,)),
        ],
        out_specs=[
            pl.BlockSpec((NUM_HEADS, ABLK, S_LEN), lambda l: (0, l, 0)),
            pl.BlockSpec((ABLK, D_MODEL), lambda l: (l, 0)),
        ],
        out_shape=[
            jax.ShapeDtypeStruct((NUM_HEADS, NUM_LOCAL, S_LEN), jnp.float32),
            jax.ShapeDtypeStruct((L_PAD, D_MODEL), jnp.float32),
        ],
        scratch_shapes=[pltpu.VMEM((D_MODEL, D_MODEL), BF)],
    )(q, k, v, Wo, bo)


def _scatter_body(idx_ref, alpha_ref, x_ref, lo_ref, out_ref):
    out_ref[...] = x_ref[...]
    alpha = alpha_ref[0]

    def step(i, _):
        idx = idx_ref[0, i]
        out_ref[pl.ds(idx, 1), :] = (out_ref[pl.ds(idx, 1), :]
                                     + alpha * lo_ref[pl.ds(i, 1), :])
        return 0

    lax.fori_loop(0, NUM_LOCAL, step, 0)


def _scatter(top_idx, alpha, x, local_out):
    return pl.pallas_call(
        _scatter_body,
        in_specs=[
            pl.BlockSpec(memory_space=pltpu.SMEM),
            pl.BlockSpec(memory_space=pltpu.SMEM),
            pl.BlockSpec(memory_space=pltpu.VMEM),
            pl.BlockSpec(memory_space=pltpu.VMEM),
        ],
        out_specs=pl.BlockSpec(memory_space=pltpu.VMEM),
        out_shape=jax.ShapeDtypeStruct((S_LEN, D_MODEL), jnp.float32),
    )(top_idx, alpha, x, local_out)


def _ffn_body(x_ref, g_ref, be_ref, w1_ref, b1_ref, w2_ref, b2_ref, o_ref,
              w1b, w2b):
    @pl.when(pl.program_id(0) == 0)
    def _():
        w1b[...] = w1_ref[...].astype(BF)
        w2b[...] = w2_ref[...].astype(BF)

    xb = x_ref[...]
    xn = _ln(xb, g_ref[...], be_ref[...]).astype(BF)
    h = jnp.dot(xn, w1b[...], preferred_element_type=jnp.float32) + b1_ref[...]
    h = 0.5 * h * (1.0 + lax.erf(h * (1.0 / math.sqrt(2.0))))
    ff = jnp.dot(h.astype(BF), w2b[...],
                 preferred_element_type=jnp.float32) + b2_ref[...]
    o_ref[...] = xb + ff


def _ffn(x_local, g2, be2, W1, b1, W2, b2):
    blk = 512
    return pl.pallas_call(
        _ffn_body,
        grid=(S_LEN // blk,),
        in_specs=[
            pl.BlockSpec((blk, D_MODEL), lambda i: (i, 0)),
            pl.BlockSpec((D_MODEL,), lambda i: (0,)),
            pl.BlockSpec((D_MODEL,), lambda i: (0,)),
            pl.BlockSpec((D_MODEL, D_FF), lambda i: (0, 0)),
            pl.BlockSpec((D_FF,), lambda i: (0,)),
            pl.BlockSpec((D_FF, D_MODEL), lambda i: (0, 0)),
            pl.BlockSpec((D_MODEL,), lambda i: (0,)),
        ],
        out_specs=pl.BlockSpec((blk, D_MODEL), lambda i: (i, 0)),
        out_shape=jax.ShapeDtypeStruct((S_LEN, D_MODEL), jnp.float32),
        scratch_shapes=[pltpu.VMEM((D_MODEL, D_FF), BF),
                        pltpu.VMEM((D_FF, D_MODEL), BF)],
    )(x_local, g2, be2, W1, b1, W2, b2)


def kernel(x, attention_rollout, Wq, bq, Wk, bk, Wv, bv, Wo, bo, W1, b1, W2, b2, g1, be1, g2, be2, alpha):
    B = x.shape[0]
    x2 = x.reshape(S_LEN, D_MODEL)
    cls = attention_rollout[0, 0, :]
    cls_col = cls.reshape(S_LEN, 1)
    cls_row = cls.reshape(1, S_LEN)

    k, v = _kv_proj(x2, g1, be1, Wk, bk, Wv, bv)
    top_idx = _topk(cls_col, cls_row)
    q = _q_proj(top_idx, x2, g1, be1, Wq, bq)
    probs, local_out = _attention(q, k, v, Wo, bo)
    x_local = _scatter(top_idx, alpha, x2, local_out)
    x_final = _ffn(x_local, g2, be2, W1, b1, W2, b2)

    return (x_final.reshape(B, S_LEN, D_MODEL),
            probs.reshape(B, NUM_HEADS, NUM_LOCAL, S_LEN))
